# baseline (speedup 1.0000x reference)
import jax
import jax.numpy as jnp
from jax.experimental import pallas as pl
from jax.experimental.pallas import tpu as pltpu

TN = 1024
DC = 1024


def _lane_chunk_sum(v, q0):
    w = v.shape[1]
    q = q0
    for k in range(w // 128):
        vk = v[:, 128 * k:128 * (k + 1)]
        p = vk * vk
        q = p if q is None else q + p
    return q


def _finish_rowsum(q):
    a = q[:, 0:8]
    for t in range(1, 16):
        a = a + q[:, 8 * t:8 * (t + 1)]
    b = a[:, 0:4] + a[:, 4:8]
    c = b[:, 0:2] + b[:, 2:4]
    return c[:, 0:1] + c[:, 1:2]


def _rowsum_sq(v):
    return _finish_rowsum(_lane_chunk_sum(v, None))


def _c2_body(c_ref, o_ref):
    o_ref[...] = _rowsum_sq(c_ref[...])


def _router_body(x_ref, c_ref, c2_ref, w_ref, a_ref, q_ref, dot_ref):
    j = pl.program_id(1)
    nj = pl.num_programs(1)

    dot_c = jax.lax.dot_general(
        x_ref[...], c_ref[...], (((1,), (1,)), ((), ())),
        preferred_element_type=jnp.float32,
    )

    @pl.when(j == 0)
    def _init():
        q_ref[...] = _lane_chunk_sum(x_ref[...], None)
        dot_ref[...] = dot_c

    @pl.when(j > 0)
    def _accum():
        q_ref[...] = _lane_chunk_sum(x_ref[...], q_ref[...])
        dot_ref[...] = dot_ref[...] + dot_c

    @pl.when(j == nj - 1)
    def _epilogue():
        dot = dot_ref[...]
        x2 = _finish_rowsum(q_ref[...])
        c2 = c2_ref[...]
        sq = jnp.maximum(x2 + c2 - 2.0 * dot, 0.0)
        dists = jnp.sqrt(sq)

        neg = -dists
        m = jnp.max(neg, axis=-1, keepdims=True)
        e = jnp.exp(neg - m)
        w_ref[...] = e / jnp.sum(e, axis=-1, keepdims=True)

        k = dists.shape[-1]
        idx = jax.lax.broadcasted_iota(jnp.int32, dists.shape, 1)
        minv = jnp.min(dists, axis=-1, keepdims=True)
        cand = jnp.where(dists == minv, idx, k)
        a_ref[...] = jnp.min(cand, axis=-1, keepdims=True)


def kernel(x, centroids):
    b, t, d = x.shape
    k = centroids.shape[0]
    n = b * t
    x_flat = x.reshape(n, d)

    c2_col = pl.pallas_call(
        _c2_body,
        out_shape=jax.ShapeDtypeStruct((k, 1), jnp.float32),
    )(centroids)
    c2_row = c2_col.reshape(1, k)

    weights, assignments = pl.pallas_call(
        _router_body,
        grid=(n // TN, d // DC),
        in_specs=[
            pl.BlockSpec((TN, DC), lambda i, j: (i, j)),
            pl.BlockSpec((k, DC), lambda i, j: (0, j)),
            pl.BlockSpec((1, k), lambda i, j: (0, 0)),
        ],
        out_specs=[
            pl.BlockSpec((TN, k), lambda i, j: (i, 0)),
            pl.BlockSpec((TN, 1), lambda i, j: (i, 0)),
        ],
        out_shape=[
            jax.ShapeDtypeStruct((n, k), jnp.float32),
            jax.ShapeDtypeStruct((n, 1), jnp.int32),
        ],
        scratch_shapes=[
            pltpu.VMEM((TN, 128), jnp.float32),
            pltpu.VMEM((TN, k), jnp.float32),
        ],
        compiler_params=pltpu.CompilerParams(
            dimension_semantics=("parallel", "arbitrary"),
        ),
    )(x_flat, centroids, c2_row)

    return weights.reshape(b, t, k), assignments.reshape(b, t)
